# baseline (speedup 1.0000x reference)
import functools

import jax
import jax.numpy as jnp
from jax.experimental import pallas as pl
from jax.experimental.pallas import tpu as pltpu

N_EMBED = 1024
NUM_EXPERTS = 8
D_FF = 4 * N_EMBED
THRESH = 1e-9
FBLK = 512
NF = D_FF // FBLK
NSTEPS = NUM_EXPERTS * NF


def _moe_body(xb_ref, Wr_ref, br_ref, W1_ref, b1_ref, W2_ref, b2_ref,
              out_ref, g_ref, hs0_ref, hs1_ref):
    s = pl.program_id(0)

    @pl.when(s == 0)
    def _prologue():
        logits = jnp.dot(xb_ref[...], Wr_ref[...].astype(jnp.bfloat16),
                         preferred_element_type=jnp.float32) + br_ref[...]
        m = jnp.max(logits, axis=-1, keepdims=True)
        p = jnp.exp(logits - m)
        gating = p / jnp.sum(p, axis=-1, keepdims=True)
        g_ref[...] = jnp.where(gating > THRESH, gating, 0.0)
        out_ref[...] = jnp.dot(g_ref[...], b2_ref[...],
                               preferred_element_type=jnp.float32)
        hs1_ref[...] = jnp.zeros(hs1_ref.shape, jnp.bfloat16)

    def _halfstep(write_ref, read_ref):
        e1 = jnp.minimum(s, NSTEPS - 1) // NF
        sel = jax.lax.broadcasted_iota(jnp.int32, (1, NUM_EXPERTS), 1) == e1
        g_e = jnp.sum(jnp.where(sel, g_ref[...], 0.0), axis=1, keepdims=True)
        w1 = W1_ref[0].astype(jnp.bfloat16)
        h = jnp.dot(xb_ref[...], w1, preferred_element_type=jnp.float32)
        h = jnp.maximum(h + b1_ref[0, 0], 0.0)
        write_ref[...] = (h * g_e).astype(jnp.bfloat16)
        w2 = W2_ref[0].astype(jnp.bfloat16)
        out_ref[...] += jnp.dot(read_ref[...], w2,
                                preferred_element_type=jnp.float32)

    @pl.when(s % 2 == 0)
    def _even():
        _halfstep(hs0_ref, hs1_ref)

    @pl.when(s % 2 == 1)
    def _odd():
        _halfstep(hs1_ref, hs0_ref)


@jax.jit
def kernel(x, Wr, br, W1, b1, W2, b2):
    B, S, D = x.shape
    T = B * S
    xb = x.reshape(T, D).astype(jnp.bfloat16)
    br2 = br.reshape(1, NUM_EXPERTS)
    b1r = b1.reshape(NUM_EXPERTS, 1, D_FF)

    def w1_idx(s):
        c = jnp.minimum(s, NSTEPS - 1)
        return (c // NF, 0, c % NF)

    def w2_idx(s):
        c = jnp.maximum(s - 1, 0)
        return (c // NF, c % NF, 0)

    out = pl.pallas_call(
        _moe_body,
        grid=(NSTEPS + 1,),
        in_specs=[
            pl.BlockSpec((T, D), lambda s: (0, 0)),
            pl.BlockSpec((D, NUM_EXPERTS), lambda s: (0, 0)),
            pl.BlockSpec((1, NUM_EXPERTS), lambda s: (0, 0)),
            pl.BlockSpec((1, D, FBLK), w1_idx),
            pl.BlockSpec((1, 1, FBLK), w1_idx),
            pl.BlockSpec((1, FBLK, D), w2_idx),
            pl.BlockSpec((NUM_EXPERTS, D), lambda s: (0, 0)),
        ],
        out_specs=pl.BlockSpec((T, D), lambda s: (0, 0)),
        out_shape=jax.ShapeDtypeStruct((T, D), jnp.float32),
        scratch_shapes=[
            pltpu.VMEM((T, NUM_EXPERTS), jnp.float32),
            pltpu.VMEM((T, FBLK), jnp.bfloat16),
            pltpu.VMEM((T, FBLK), jnp.bfloat16),
        ],
        compiler_params=pltpu.CompilerParams(
            dimension_semantics=("arbitrary",),
        ),
    )(xb, Wr, br2, W1, b1r, W2, b2)
    return out.reshape(B, S, D)


# speedup vs baseline: 1.0338x; 1.0338x over previous
import functools

import jax
import jax.numpy as jnp
from jax.experimental import pallas as pl
from jax.experimental.pallas import tpu as pltpu

N_EMBED = 1024
NUM_EXPERTS = 8
D_FF = 4 * N_EMBED
THRESH = 1e-9
FBLK = 512
NF = D_FF // FBLK


def _moe_body(xb_ref, Wr_ref, br_ref, W1_ref, b1_ref, W2_ref, b2_ref,
              out_ref, g_ref):
    e = pl.program_id(0)
    f = pl.program_id(1)

    @pl.when(jnp.logical_and(e == 0, f == 0))
    def _prologue():
        logits = jnp.dot(xb_ref[...], Wr_ref[...].astype(jnp.bfloat16),
                         preferred_element_type=jnp.float32) + br_ref[...]
        m = jnp.max(logits, axis=-1, keepdims=True)
        p = jnp.exp(logits - m)
        gating = p / jnp.sum(p, axis=-1, keepdims=True)
        g_ref[...] = jnp.where(gating > THRESH, gating, 0.0)
        out_ref[...] = jnp.dot(g_ref[...], b2_ref[...],
                               preferred_element_type=jnp.float32)

    sel = jax.lax.broadcasted_iota(jnp.int32, (1, NUM_EXPERTS), 1) == e
    g_e = jnp.sum(jnp.where(sel, g_ref[...], 0.0), axis=1, keepdims=True)

    w1 = W1_ref[0].astype(jnp.bfloat16)
    h = jnp.dot(xb_ref[...], w1,
                preferred_element_type=jnp.float32).astype(jnp.bfloat16)
    h = jnp.maximum(h + b1_ref[0, 0].astype(jnp.bfloat16),
                    jnp.bfloat16(0.0))
    hs = h * g_e.astype(jnp.bfloat16)
    w2 = W2_ref[0].astype(jnp.bfloat16)
    out_ref[...] += jnp.dot(hs, w2, preferred_element_type=jnp.float32)


@jax.jit
def kernel(x, Wr, br, W1, b1, W2, b2):
    B, S, D = x.shape
    T = B * S
    xb = x.reshape(T, D).astype(jnp.bfloat16)
    br2 = br.reshape(1, NUM_EXPERTS)
    b1r = b1.reshape(NUM_EXPERTS, 1, D_FF)

    out = pl.pallas_call(
        _moe_body,
        grid=(NUM_EXPERTS, NF),
        in_specs=[
            pl.BlockSpec((T, D), lambda e, f: (0, 0)),
            pl.BlockSpec((D, NUM_EXPERTS), lambda e, f: (0, 0)),
            pl.BlockSpec((1, NUM_EXPERTS), lambda e, f: (0, 0)),
            pl.BlockSpec((1, D, FBLK), lambda e, f: (e, 0, f)),
            pl.BlockSpec((1, 1, FBLK), lambda e, f: (e, 0, f)),
            pl.BlockSpec((1, FBLK, D), lambda e, f: (e, f, 0)),
            pl.BlockSpec((NUM_EXPERTS, D), lambda e, f: (0, 0)),
        ],
        out_specs=pl.BlockSpec((T, D), lambda e, f: (0, 0)),
        out_shape=jax.ShapeDtypeStruct((T, D), jnp.float32),
        scratch_shapes=[pltpu.VMEM((T, NUM_EXPERTS), jnp.float32)],
        compiler_params=pltpu.CompilerParams(
            dimension_semantics=("arbitrary", "arbitrary"),
        ),
    )(xb, Wr, br2, W1, b1r, W2, b2)
    return out.reshape(B, S, D)
